# baseline (speedup 1.0000x reference)
import jax
import jax.numpy as jnp
from jax import lax
from jax.experimental import pallas as pl
from jax.experimental.pallas import tpu as pltpu
from jax.experimental.pallas import tpu_sc as plsc

N = 100000
E = 3200000
N_PAD = 100352
NC = 2
NS = 16
NW = NC * NS

BLK = 128
SUP = 40
EB = E // BLK
NCH = EB // SUP

COVERAGE_W = 5.0
TOUR_W = 3.0
DEPOT_W = 2.0
SIM_W = 0.3
FOCAL_ALPHA = 0.25
FOCAL_GAMMA = 2.0


TC_ROWS = 1000
TC_STEPS = EB // TC_ROWS


def _focal_body(preds_ref, y_ref, out_ref, probs_ref, acc):
  i = pl.program_id(0)

  @pl.when(i == 0)
  def _():
    acc[0] = 0.0

  x = preds_ref[...]
  t = y_ref[...]
  es = jnp.exp(-jnp.abs(x))
  u = 1.0 / (1.0 + es)
  bce = jnp.maximum(x, 0.0) - x * t + jnp.log1p(es)
  probs = jnp.where(x >= 0.0, u, es * u)
  probs_ref[...] = probs
  p_t = probs * t + (1.0 - probs) * (1.0 - t)
  alpha_t = FOCAL_ALPHA * t + (1.0 - FOCAL_ALPHA) * (1.0 - t)
  w = 1.0 - p_t
  acc[0] += jnp.sum(alpha_t * (w * w) * bce)

  @pl.when(i == TC_STEPS - 1)
  def _():
    out_ref[0, 0] = acc[0]


def _tc_focal(preds2d, y2d):
  return pl.pallas_call(
      _focal_body,
      grid=(TC_STEPS,),
      in_specs=[
          pl.BlockSpec((TC_ROWS, 128), lambda i: (i, 0)),
          pl.BlockSpec((TC_ROWS, 128), lambda i: (i, 0)),
      ],
      out_specs=[
          pl.BlockSpec(memory_space=pltpu.SMEM),
          pl.BlockSpec((TC_ROWS, 128), lambda i: (i, 0)),
      ],
      out_shape=[
          jax.ShapeDtypeStruct((1, 1), jnp.float32),
          jax.ShapeDtypeStruct((EB, BLK), jnp.float32),
      ],
      scratch_shapes=[pltpu.SMEM((1,), jnp.float32)],
  )(preds2d, y2d)


def _sc_body(probs_hbm, idx_hbm, out_hbm,
             acc, pbuf0, pbuf1, ibuf0, ibuf1, sem0, sem1):
  c = lax.axis_index("c")
  s = lax.axis_index("s")
  wid = c * NS + s
  sel = 1 - c

  def _zero(i, _):
    acc[pl.ds(i * 16, 16)] = jnp.zeros((16,), jnp.float32)
    return 0
  lax.fori_loop(0, N_PAD // 16, _zero, 0, unroll=16)

  g0 = (s * NCH) // NS
  g1 = ((s + 1) * NCH) // NS

  def _load(g, pbuf, ibuf, sem):
    row = g * SUP
    pltpu.async_copy(probs_hbm.at[pl.ds(row, SUP), :], pbuf, sem)
    pltpu.async_copy(idx_hbm.at[sel, pl.ds(row * BLK, SUP * BLK)], ibuf, sem)

  def _wait(pbuf, ibuf, sem):
    pltpu.make_async_copy(probs_hbm.at[pl.ds(0, SUP), :], pbuf, sem).wait()
    pltpu.make_async_copy(idx_hbm.at[0, pl.ds(0, SUP * BLK)], ibuf, sem).wait()

  def _compute(pbuf, ibuf):
    def _row(r, _):
      for j in range(BLK // 16):
        p = pbuf[r, pl.ds(j * 16, 16)]
        ids = ibuf[pl.ds(r * BLK + j * 16, 16)]
        plsc.addupdate_scatter(acc, [ids], p)
      return 0
    lax.fori_loop(0, SUP, _row, 0, unroll=2)

  _load(g0, pbuf0, ibuf0, sem0)

  def _pair(k, _):
    a = g0 + 2 * k
    b = a + 1

    @pl.when(b < g1)
    def _():
      _load(b, pbuf1, ibuf1, sem1)

    _wait(pbuf0, ibuf0, sem0)
    _compute(pbuf0, ibuf0)

    @pl.when(a + 2 < g1)
    def _():
      _load(a + 2, pbuf0, ibuf0, sem0)

    @pl.when(b < g1)
    def _():
      _wait(pbuf1, ibuf1, sem1)
      _compute(pbuf1, ibuf1)
    return 0

  lax.fori_loop(0, (g1 - g0 + 1) // 2, _pair, 0)

  pltpu.sync_copy(acc, out_hbm.at[wid, 0])


def _sc_segment_sums(probs2d, idx3d):
  mesh = plsc.VectorSubcoreMesh(core_axis_name="c", subcore_axis_name="s")
  f = pl.kernel(
      _sc_body,
      out_type=jax.ShapeDtypeStruct((NW, 1, N_PAD), jnp.float32),
      mesh=mesh,
      compiler_params=pltpu.CompilerParams(needs_layout_passes=False),
      scratch_types=[
          pltpu.VMEM((N_PAD,), jnp.float32),
          pltpu.VMEM((SUP, BLK), jnp.float32),
          pltpu.VMEM((SUP, BLK), jnp.float32),
          pltpu.VMEM((SUP * BLK,), jnp.int32),
          pltpu.VMEM((SUP * BLK,), jnp.int32),
          pltpu.SemaphoreType.DMA,
          pltpu.SemaphoreType.DMA,
      ],
  )
  return f(probs2d, idx3d)


CB_ROWS = 112
CB_STEPS = N_PAD // 128 // CB_ROWS


def _combine_body(planes_ref, focal_ref, out_ref, acc):
  j = pl.program_id(0)

  @pl.when(j == 0)
  def _():
    acc[0] = 0.0
    acc[1] = 0.0
    acc[2] = 0.0

  in_s = planes_ref[0]
  out_s = planes_ref[NS]
  for k in range(1, NS):
    in_s = in_s + planes_ref[k]
    out_s = out_s + planes_ref[NS + k]

  n = (lax.broadcasted_iota(jnp.int32, (CB_ROWS, 128), 0) * 128
       + lax.broadcasted_iota(jnp.int32, (CB_ROWS, 128), 1)
       + j * (CB_ROWS * 128))
  customer = jnp.logical_and(n >= 1, n < N)
  zero = jnp.zeros_like(in_s)
  acc[0] += jnp.sum(jnp.where(customer, (in_s - 1.0) ** 2, zero)
                    + jnp.where(customer, (out_s - 1.0) ** 2, zero))
  diff = in_s - out_s
  acc[1] += jnp.sum(diff * diff)

  @pl.when(j == 0)
  def _():
    acc[2] = diff[0, 0] * diff[0, 0]

  @pl.when(j == CB_STEPS - 1)
  def _():
    out_ref[0, 0] = (COVERAGE_W * acc[0] / (2.0 * (N - 1))
                     + TOUR_W * acc[1] / N
                     + DEPOT_W * acc[2]
                     + SIM_W * focal_ref[0, 0] / E)


def _tc_combine(planes3, focal):
  return pl.pallas_call(
      _combine_body,
      grid=(CB_STEPS,),
      in_specs=[
          pl.BlockSpec((NW, CB_ROWS, 128), lambda j: (0, j, 0)),
          pl.BlockSpec(memory_space=pltpu.SMEM),
      ],
      out_specs=pl.BlockSpec(memory_space=pltpu.SMEM),
      out_shape=jax.ShapeDtypeStruct((1, 1), jnp.float32),
      scratch_shapes=[pltpu.SMEM((4,), jnp.float32)],
  )(planes3, focal)


def kernel(edge_predictions, edge_index, y_edges, num_nodes):
  preds2d = edge_predictions.reshape(EB, BLK)
  y2d = y_edges.reshape(EB, BLK)

  focal, probs2d = _tc_focal(preds2d, y2d)
  planes = _sc_segment_sums(probs2d, edge_index)
  planes3 = planes.reshape(NW, N_PAD // 128, 128)
  total = _tc_combine(planes3, focal)
  return total.reshape(())


# speedup vs baseline: 1.0120x; 1.0120x over previous
import jax
import jax.numpy as jnp
from jax import lax
from jax.experimental import pallas as pl
from jax.experimental.pallas import tpu as pltpu
from jax.experimental.pallas import tpu_sc as plsc

N = 100000
E = 3200000
N_PAD = 100352
NC = 2
NS = 16
NW = NC * NS

BLK = 128
SUP = 40
EB = E // BLK
NCH = EB // SUP

COVERAGE_W = 5.0
TOUR_W = 3.0
DEPOT_W = 2.0
SIM_W = 0.3
FOCAL_ALPHA = 0.25
FOCAL_GAMMA = 2.0


TC_ROWS = 1000
TC_STEPS = EB // TC_ROWS


def _focal_body(preds_ref, y_ref, out_ref, probs_ref, acc):
  i = pl.program_id(0)

  @pl.when(i == 0)
  def _():
    acc[0] = 0.0

  x = preds_ref[...]
  t = y_ref[...]
  bce = jnp.maximum(x, 0.0) - x * t + jnp.log1p(jnp.exp(-jnp.abs(x)))
  probs = jax.nn.sigmoid(x)
  probs_ref[...] = probs
  p_t = probs * t + (1.0 - probs) * (1.0 - t)
  alpha_t = FOCAL_ALPHA * t + (1.0 - FOCAL_ALPHA) * (1.0 - t)
  w = 1.0 - p_t
  acc[0] += jnp.sum(alpha_t * (w * w) * bce)

  @pl.when(i == TC_STEPS - 1)
  def _():
    out_ref[0, 0] = acc[0]


def _tc_focal(preds2d, y2d):
  return pl.pallas_call(
      _focal_body,
      grid=(TC_STEPS,),
      in_specs=[
          pl.BlockSpec((TC_ROWS, 128), lambda i: (i, 0)),
          pl.BlockSpec((TC_ROWS, 128), lambda i: (i, 0)),
      ],
      out_specs=[
          pl.BlockSpec(memory_space=pltpu.SMEM),
          pl.BlockSpec((TC_ROWS, 128), lambda i: (i, 0)),
      ],
      out_shape=[
          jax.ShapeDtypeStruct((1, 1), jnp.float32),
          jax.ShapeDtypeStruct((EB, BLK), jnp.float32),
      ],
      scratch_shapes=[pltpu.SMEM((1,), jnp.float32)],
  )(preds2d, y2d)


def _sc_body(probs_hbm, idx_hbm, out_hbm,
             acc, pbuf0, pbuf1, ibuf0, ibuf1, sem0, sem1):
  c = lax.axis_index("c")
  s = lax.axis_index("s")
  wid = c * NS + s
  sel = 1 - c

  def _zero(i, _):
    acc[pl.ds(i * 16, 16)] = jnp.zeros((16,), jnp.float32)
    return 0
  lax.fori_loop(0, N_PAD // 16, _zero, 0, unroll=16)

  g0 = (s * NCH) // NS
  g1 = ((s + 1) * NCH) // NS

  def _load(g, pbuf, ibuf, sem):
    row = g * SUP
    pltpu.async_copy(probs_hbm.at[pl.ds(row, SUP), :], pbuf, sem)
    pltpu.async_copy(idx_hbm.at[sel, pl.ds(row * BLK, SUP * BLK)], ibuf, sem)

  def _wait(pbuf, ibuf, sem):
    pltpu.make_async_copy(probs_hbm.at[pl.ds(0, SUP), :], pbuf, sem).wait()
    pltpu.make_async_copy(idx_hbm.at[0, pl.ds(0, SUP * BLK)], ibuf, sem).wait()

  def _compute(pbuf, ibuf):
    def _row(r, _):
      for j in range(BLK // 16):
        p = pbuf[r, pl.ds(j * 16, 16)]
        ids = ibuf[pl.ds(r * BLK + j * 16, 16)]
        plsc.addupdate_scatter(acc, [ids], p)
      return 0
    lax.fori_loop(0, SUP, _row, 0, unroll=4)

  _load(g0, pbuf0, ibuf0, sem0)

  def _pair(k, _):
    a = g0 + 2 * k
    b = a + 1

    @pl.when(b < g1)
    def _():
      _load(b, pbuf1, ibuf1, sem1)

    _wait(pbuf0, ibuf0, sem0)
    _compute(pbuf0, ibuf0)

    @pl.when(a + 2 < g1)
    def _():
      _load(a + 2, pbuf0, ibuf0, sem0)

    @pl.when(b < g1)
    def _():
      _wait(pbuf1, ibuf1, sem1)
      _compute(pbuf1, ibuf1)
    return 0

  lax.fori_loop(0, (g1 - g0 + 1) // 2, _pair, 0)

  pltpu.sync_copy(acc, out_hbm.at[wid, 0])


def _sc_segment_sums(probs2d, idx3d):
  mesh = plsc.VectorSubcoreMesh(core_axis_name="c", subcore_axis_name="s")
  f = pl.kernel(
      _sc_body,
      out_type=jax.ShapeDtypeStruct((NW, 1, N_PAD), jnp.float32),
      mesh=mesh,
      compiler_params=pltpu.CompilerParams(needs_layout_passes=False),
      scratch_types=[
          pltpu.VMEM((N_PAD,), jnp.float32),
          pltpu.VMEM((SUP, BLK), jnp.float32),
          pltpu.VMEM((SUP, BLK), jnp.float32),
          pltpu.VMEM((SUP * BLK,), jnp.int32),
          pltpu.VMEM((SUP * BLK,), jnp.int32),
          pltpu.SemaphoreType.DMA,
          pltpu.SemaphoreType.DMA,
      ],
  )
  return f(probs2d, idx3d)


CB_ROWS = 112
CB_STEPS = N_PAD // 128 // CB_ROWS


def _combine_body(planes_ref, focal_ref, out_ref, acc):
  j = pl.program_id(0)

  @pl.when(j == 0)
  def _():
    acc[0] = 0.0
    acc[1] = 0.0
    acc[2] = 0.0

  in_s = planes_ref[0]
  out_s = planes_ref[NS]
  for k in range(1, NS):
    in_s = in_s + planes_ref[k]
    out_s = out_s + planes_ref[NS + k]

  n = (lax.broadcasted_iota(jnp.int32, (CB_ROWS, 128), 0) * 128
       + lax.broadcasted_iota(jnp.int32, (CB_ROWS, 128), 1)
       + j * (CB_ROWS * 128))
  customer = jnp.logical_and(n >= 1, n < N)
  zero = jnp.zeros_like(in_s)
  acc[0] += jnp.sum(jnp.where(customer, (in_s - 1.0) ** 2, zero)
                    + jnp.where(customer, (out_s - 1.0) ** 2, zero))
  diff = in_s - out_s
  acc[1] += jnp.sum(diff * diff)

  @pl.when(j == 0)
  def _():
    acc[2] = diff[0, 0] * diff[0, 0]

  @pl.when(j == CB_STEPS - 1)
  def _():
    out_ref[0, 0] = (COVERAGE_W * acc[0] / (2.0 * (N - 1))
                     + TOUR_W * acc[1] / N
                     + DEPOT_W * acc[2]
                     + SIM_W * focal_ref[0, 0] / E)


def _tc_combine(planes3, focal):
  return pl.pallas_call(
      _combine_body,
      grid=(CB_STEPS,),
      in_specs=[
          pl.BlockSpec((NW, CB_ROWS, 128), lambda j: (0, j, 0)),
          pl.BlockSpec(memory_space=pltpu.SMEM),
      ],
      out_specs=pl.BlockSpec(memory_space=pltpu.SMEM),
      out_shape=jax.ShapeDtypeStruct((1, 1), jnp.float32),
      scratch_shapes=[pltpu.SMEM((4,), jnp.float32)],
  )(planes3, focal)


def kernel(edge_predictions, edge_index, y_edges, num_nodes):
  preds2d = edge_predictions.reshape(EB, BLK)
  y2d = y_edges.reshape(EB, BLK)

  focal, probs2d = _tc_focal(preds2d, y2d)
  planes = _sc_segment_sums(probs2d, edge_index)
  planes3 = planes.reshape(NW, N_PAD // 128, 128)
  total = _tc_combine(planes3, focal)
  return total.reshape(())


# speedup vs baseline: 1.0372x; 1.0249x over previous
import jax
import jax.numpy as jnp
from jax import lax
from jax.experimental import pallas as pl
from jax.experimental.pallas import tpu as pltpu
from jax.experimental.pallas import tpu_sc as plsc

N = 100000
E = 3200000
N_PAD = 100352
NC = 2
NS = 16
NW = NC * NS

BLK = 128
SUP = 40
EB = E // BLK
NCH = EB // SUP

COVERAGE_W = 5.0
TOUR_W = 3.0
DEPOT_W = 2.0
SIM_W = 0.3
FOCAL_ALPHA = 0.25
FOCAL_GAMMA = 2.0


TC_ROWS = 5000
TC_STEPS = EB // TC_ROWS


def _focal_body(preds_ref, y_ref, out_ref, probs_ref, acc):
  i = pl.program_id(0)

  @pl.when(i == 0)
  def _():
    acc[0] = 0.0

  x = preds_ref[...]
  t = y_ref[...]
  bce = jnp.maximum(x, 0.0) - x * t + jnp.log1p(jnp.exp(-jnp.abs(x)))
  probs = jax.nn.sigmoid(x)
  probs_ref[...] = probs
  p_t = probs * t + (1.0 - probs) * (1.0 - t)
  alpha_t = FOCAL_ALPHA * t + (1.0 - FOCAL_ALPHA) * (1.0 - t)
  w = 1.0 - p_t
  acc[0] += jnp.sum(alpha_t * (w * w) * bce)

  @pl.when(i == TC_STEPS - 1)
  def _():
    out_ref[0, 0] = acc[0]


def _tc_focal(preds2d, y2d):
  return pl.pallas_call(
      _focal_body,
      grid=(TC_STEPS,),
      in_specs=[
          pl.BlockSpec((TC_ROWS, 128), lambda i: (i, 0)),
          pl.BlockSpec((TC_ROWS, 128), lambda i: (i, 0)),
      ],
      out_specs=[
          pl.BlockSpec(memory_space=pltpu.SMEM),
          pl.BlockSpec((TC_ROWS, 128), lambda i: (i, 0)),
      ],
      out_shape=[
          jax.ShapeDtypeStruct((1, 1), jnp.float32),
          jax.ShapeDtypeStruct((EB, BLK), jnp.float32),
      ],
      scratch_shapes=[pltpu.SMEM((1,), jnp.float32)],
  )(preds2d, y2d)


def _sc_body(probs_hbm, idx_hbm, out_hbm,
             acc, pbuf0, pbuf1, ibuf0, ibuf1, sem0, sem1):
  c = lax.axis_index("c")
  s = lax.axis_index("s")
  wid = c * NS + s
  sel = 1 - c

  def _zero(i, _):
    acc[pl.ds(i * 16, 16)] = jnp.zeros((16,), jnp.float32)
    return 0
  lax.fori_loop(0, N_PAD // 16, _zero, 0, unroll=16)

  g0 = (s * NCH) // NS
  g1 = ((s + 1) * NCH) // NS

  def _load(g, pbuf, ibuf, sem):
    row = g * SUP
    pltpu.async_copy(probs_hbm.at[pl.ds(row, SUP), :], pbuf, sem)
    pltpu.async_copy(idx_hbm.at[sel, pl.ds(row * BLK, SUP * BLK)], ibuf, sem)

  def _wait(pbuf, ibuf, sem):
    pltpu.make_async_copy(probs_hbm.at[pl.ds(0, SUP), :], pbuf, sem).wait()
    pltpu.make_async_copy(idx_hbm.at[0, pl.ds(0, SUP * BLK)], ibuf, sem).wait()

  def _compute(pbuf, ibuf):
    def _row(r, _):
      for j in range(BLK // 16):
        p = pbuf[r, pl.ds(j * 16, 16)]
        ids = ibuf[pl.ds(r * BLK + j * 16, 16)]
        plsc.addupdate_scatter(acc, [ids], p)
      return 0
    lax.fori_loop(0, SUP, _row, 0, unroll=4)

  _load(g0, pbuf0, ibuf0, sem0)

  def _pair(k, _):
    a = g0 + 2 * k
    b = a + 1

    @pl.when(b < g1)
    def _():
      _load(b, pbuf1, ibuf1, sem1)

    _wait(pbuf0, ibuf0, sem0)
    _compute(pbuf0, ibuf0)

    @pl.when(a + 2 < g1)
    def _():
      _load(a + 2, pbuf0, ibuf0, sem0)

    @pl.when(b < g1)
    def _():
      _wait(pbuf1, ibuf1, sem1)
      _compute(pbuf1, ibuf1)
    return 0

  lax.fori_loop(0, (g1 - g0 + 1) // 2, _pair, 0)

  pltpu.sync_copy(acc, out_hbm.at[wid, 0])


def _sc_segment_sums(probs2d, idx3d):
  mesh = plsc.VectorSubcoreMesh(core_axis_name="c", subcore_axis_name="s")
  f = pl.kernel(
      _sc_body,
      out_type=jax.ShapeDtypeStruct((NW, 1, N_PAD), jnp.float32),
      mesh=mesh,
      compiler_params=pltpu.CompilerParams(needs_layout_passes=False),
      scratch_types=[
          pltpu.VMEM((N_PAD,), jnp.float32),
          pltpu.VMEM((SUP, BLK), jnp.float32),
          pltpu.VMEM((SUP, BLK), jnp.float32),
          pltpu.VMEM((SUP * BLK,), jnp.int32),
          pltpu.VMEM((SUP * BLK,), jnp.int32),
          pltpu.SemaphoreType.DMA,
          pltpu.SemaphoreType.DMA,
      ],
  )
  return f(probs2d, idx3d)


CB_ROWS = 112
CB_STEPS = N_PAD // 128 // CB_ROWS


def _combine_body(planes_ref, focal_ref, out_ref, acc):
  j = pl.program_id(0)

  @pl.when(j == 0)
  def _():
    acc[0] = 0.0
    acc[1] = 0.0
    acc[2] = 0.0

  in_s = planes_ref[0]
  out_s = planes_ref[NS]
  for k in range(1, NS):
    in_s = in_s + planes_ref[k]
    out_s = out_s + planes_ref[NS + k]

  n = (lax.broadcasted_iota(jnp.int32, (CB_ROWS, 128), 0) * 128
       + lax.broadcasted_iota(jnp.int32, (CB_ROWS, 128), 1)
       + j * (CB_ROWS * 128))
  customer = jnp.logical_and(n >= 1, n < N)
  zero = jnp.zeros_like(in_s)
  acc[0] += jnp.sum(jnp.where(customer, (in_s - 1.0) ** 2, zero)
                    + jnp.where(customer, (out_s - 1.0) ** 2, zero))
  diff = in_s - out_s
  acc[1] += jnp.sum(diff * diff)

  @pl.when(j == 0)
  def _():
    acc[2] = diff[0, 0] * diff[0, 0]

  @pl.when(j == CB_STEPS - 1)
  def _():
    out_ref[0, 0] = (COVERAGE_W * acc[0] / (2.0 * (N - 1))
                     + TOUR_W * acc[1] / N
                     + DEPOT_W * acc[2]
                     + SIM_W * focal_ref[0, 0] / E)


def _tc_combine(planes3, focal):
  return pl.pallas_call(
      _combine_body,
      grid=(CB_STEPS,),
      in_specs=[
          pl.BlockSpec((NW, CB_ROWS, 128), lambda j: (0, j, 0)),
          pl.BlockSpec(memory_space=pltpu.SMEM),
      ],
      out_specs=pl.BlockSpec(memory_space=pltpu.SMEM),
      out_shape=jax.ShapeDtypeStruct((1, 1), jnp.float32),
      scratch_shapes=[pltpu.SMEM((4,), jnp.float32)],
  )(planes3, focal)


def kernel(edge_predictions, edge_index, y_edges, num_nodes):
  preds2d = edge_predictions.reshape(EB, BLK)
  y2d = y_edges.reshape(EB, BLK)

  focal, probs2d = _tc_focal(preds2d, y2d)
  planes = _sc_segment_sums(probs2d, edge_index)
  planes3 = planes.reshape(NW, N_PAD // 128, 128)
  total = _tc_combine(planes3, focal)
  return total.reshape(())
